# baseline (speedup 1.0000x reference)
import functools

import jax
import jax.numpy as jnp
from jax import lax
from jax.experimental import pallas as pl
from jax.experimental.pallas import tpu as pltpu
from jax.experimental.pallas import tpu_sc as plsc

N = 10000
D = 128
E = 320000

NC = 2
NS = 16
NW = NC * NS

K = 128
CH = 80
CHH = 40
EPT = CH * K
EPAD = NW * EPT
NPAD = 10112
RPT = NPAD // NS

BLK = 1000
GRID = N // BLK


def _agg_body(h_hbm, srcs_hbm, dsts_hbm, agg_hbm,
              sidx_v, didx_v, rows0_v, rows1_v, sem0, sem1, ssem0, ssem1,
              agg_s):
  c = lax.axis_index("c")
  s = lax.axis_index("s")
  wid = s * NC + c
  r0 = pl.multiple_of(s * RPT, 8)

  def zstore(i, carry):
    rows0_v[i // 8, pl.ds((i % 8) * 16, 16)] = jnp.zeros((16,), jnp.float32)
    return carry

  with jax.named_scope("agg_zero_init"):
    lax.fori_loop(0, K * 8, zstore, 0)
    for t in range(5):
      nrows = K if t < 4 else RPT - 4 * K
      pltpu.sync_copy(rows0_v.at[pl.ds(0, nrows)],
                      agg_s.at[pl.ds(r0 + t * K, nrows)])
    plsc.subcore_barrier()

  def gather(j, buf, sem):
    return pltpu.make_async_copy(h_hbm.at[sidx_v.at[j]], buf, sem)

  def scatter(j, buf):
    pltpu.sync_copy(buf, agg_s.at[didx_v.at[j]], add=True)

  with jax.named_scope("agg_edge_loop"):
    for half in range(CH // CHH):
      base = half * CHH
      pltpu.sync_copy(srcs_hbm.at[wid, pl.ds(base, CHH)], sidx_v)
      pltpu.sync_copy(dsts_hbm.at[wid, pl.ds(base, CHH)], didx_v)

      def chunk(j, carry):
        gather(j, rows0_v, sem0).start()
        gather(j, rows0_v, sem0).wait()
        scatter(j, rows0_v)
        return carry

      lax.fori_loop(0, CHH, chunk, 0)

  with jax.named_scope("agg_barrier2"):
    plsc.subcore_barrier()
  with jax.named_scope("agg_copy_out"):
    pltpu.sync_copy(agg_s.at[pl.ds(r0, RPT)], agg_hbm.at[c, pl.ds(r0, RPT)])


def _cnt_body(dsts_hbm, z128_hbm, o128_hbm, cnt_hbm,
              didx_v, ones_v, cnt_s):
  c = lax.axis_index("c")
  s = lax.axis_index("s")
  wid = s * NC + c
  r0 = pl.multiple_of(s * RPT, 8)

  pltpu.sync_copy(dsts_hbm.at[wid], didx_v)
  pltpu.sync_copy(z128_hbm.at[pl.ds(r0, RPT)], cnt_s.at[pl.ds(r0, RPT)])
  pltpu.sync_copy(o128_hbm, ones_v)
  plsc.subcore_barrier()

  def chunk(j, carry):
    pltpu.sync_copy(ones_v, cnt_s.at[didx_v.at[j]], add=True)
    return carry

  lax.fori_loop(0, CH, chunk, 0)
  plsc.subcore_barrier()
  pltpu.sync_copy(cnt_s.at[pl.ds(r0, RPT)], cnt_hbm.at[c, pl.ds(r0, RPT)])


@functools.lru_cache(maxsize=None)
def _sc_kernels():
  mesh = plsc.VectorSubcoreMesh(
      core_axis_name="c", subcore_axis_name="s",
      num_cores=NC, num_subcores=NS)
  sc_agg = pl.kernel(
      _agg_body,
      out_type=jax.ShapeDtypeStruct((NC, NPAD, D), jnp.float32),
      mesh=mesh,
      scratch_types=[
          pltpu.VMEM((CHH, K), jnp.int32),
          pltpu.VMEM((CHH, K), jnp.int32),
          pltpu.VMEM((K, D), jnp.float32),
          pltpu.VMEM((K, D), jnp.float32),
          pltpu.SemaphoreType.DMA,
          pltpu.SemaphoreType.DMA,
          pltpu.SemaphoreType.DMA,
          pltpu.SemaphoreType.DMA,
          pltpu.VMEM_SHARED((NPAD, D), jnp.float32),
      ])
  sc_cnt = pl.kernel(
      _cnt_body,
      out_type=jax.ShapeDtypeStruct((NC, NPAD, D), jnp.float32),
      mesh=mesh,
      scratch_types=[
          pltpu.VMEM((CH, K), jnp.int32),
          pltpu.VMEM((K, D), jnp.float32),
          pltpu.VMEM_SHARED((NPAD, D), jnp.float32),
      ])
  return sc_agg, sc_cnt


def _mean(agg_ref, cnt_ref):
  inv = 1.0 / jnp.maximum(cnt_ref[0, :, 0:1] + cnt_ref[1, :, 0:1], 1.0)
  return (agg_ref[0] + agg_ref[1]) * inv


def _layer_body(agg_ref, cnt_ref, h_ref, wl_ref, bl_ref, wr_ref, out_ref):
  mean = _mean(agg_ref, cnt_ref)
  out_ref[...] = jnp.maximum(
      jnp.dot(mean, wl_ref[...], preferred_element_type=jnp.float32)
      + bl_ref[...]
      + jnp.dot(h_ref[...], wr_ref[...], preferred_element_type=jnp.float32),
      0.0)


def _final_body(agg_ref, cnt_ref, h2_ref, h1_ref, rr_ref,
                wl_ref, bl_ref, wr_ref, ws1_ref, ws2_ref, ws3_ref,
                bs_ref, alpha_ref, out_ref):
  mean = _mean(agg_ref, cnt_ref)
  h3 = jnp.maximum(
      jnp.dot(mean, wl_ref[...], preferred_element_type=jnp.float32)
      + bl_ref[...]
      + jnp.dot(h2_ref[...], wr_ref[...], preferred_element_type=jnp.float32),
      0.0)
  gnn = (jnp.dot(h1_ref[...], ws1_ref[...], preferred_element_type=jnp.float32)
         + jnp.dot(h2_ref[...], ws2_ref[...], preferred_element_type=jnp.float32)
         + jnp.dot(h3, ws3_ref[...], preferred_element_type=jnp.float32)
         + bs_ref[0, 0])
  a = 1.0 / (1.0 + jnp.exp(-alpha_ref[0, 0]))
  out_ref[...] = a * rr_ref[...] + (1.0 - a) * gnn


_agg_spec = pl.BlockSpec((NC, BLK, D), lambda i: (0, i, 0))
_cnt_spec = pl.BlockSpec((NC, BLK, D), lambda i: (0, i, 0))
_row_spec = pl.BlockSpec((BLK, D), lambda i: (i, 0))
_w_spec = pl.BlockSpec((D, D), lambda i: (0, 0))
_b_spec = pl.BlockSpec((1, D), lambda i: (0, 0))
_ws_spec = pl.BlockSpec((D, 1), lambda i: (0, 0))
_s1_spec = pl.BlockSpec((BLK, 1), lambda i: (i, 0))
_sc_spec = pl.BlockSpec((1, 1), lambda i: (0, 0))

_layer_call = pl.pallas_call(
    _layer_body,
    grid=(GRID,),
    in_specs=[_agg_spec, _cnt_spec, _row_spec, _w_spec, _b_spec, _w_spec],
    out_specs=_row_spec,
    out_shape=jax.ShapeDtypeStruct((N, D), jnp.float32),
)

_final_call = pl.pallas_call(
    _final_body,
    grid=(GRID,),
    in_specs=[_agg_spec, _cnt_spec, _row_spec, _row_spec, _s1_spec,
              _w_spec, _b_spec, _w_spec, _ws_spec, _ws_spec, _ws_spec,
              _sc_spec, _sc_spec],
    out_specs=_s1_spec,
    out_shape=jax.ShapeDtypeStruct((N, 1), jnp.float32),
)


def kernel(x, edge_index, reranker_scores, Wl0, bl0, Wr0, Wl1, bl1, Wr1,
           Wl2, bl2, Wr2, Ws, bs, alpha):
  src = edge_index[0]
  dst = edge_index[1]
  pad = EPAD - E
  srcs = jnp.concatenate([src, jnp.zeros((pad,), jnp.int32)]).reshape(NW, CH, K)
  dsts = jnp.concatenate([dst, jnp.full((pad,), N, jnp.int32)]).reshape(NW, CH, K)
  z128 = jnp.zeros((NPAD, D), jnp.float32)
  o128 = jnp.ones((K, D), jnp.float32)

  _sc_agg, _sc_cnt = _sc_kernels()
  cnt = _sc_cnt(dsts, z128, o128)
  agg0 = _sc_agg(x, srcs, dsts)
  h1 = _layer_call(agg0, cnt, x, Wl0, bl0.reshape(1, D), Wr0)
  agg1 = _sc_agg(h1, srcs, dsts)
  h2 = _layer_call(agg1, cnt, h1, Wl1, bl1.reshape(1, D), Wr1)
  agg2 = _sc_agg(h2, srcs, dsts)
  out = _final_call(
      agg2, cnt, h2, h1, reranker_scores.reshape(N, 1),
      Wl2, bl2.reshape(1, D), Wr2,
      Ws[0:D], Ws[D:2 * D], Ws[2 * D:3 * D],
      bs.reshape(1, 1), alpha.reshape(1, 1))
  return out.reshape(N)


# speedup vs baseline: 1.0038x; 1.0038x over previous
import functools

import jax
import jax.numpy as jnp
from jax import lax
from jax.experimental import pallas as pl
from jax.experimental.pallas import tpu as pltpu
from jax.experimental.pallas import tpu_sc as plsc

N = 10000
D = 128
E = 320000

NC = 2
NS = 16
NW = NC * NS

K = 128
CH = 80
CHH = 40
EPT = CH * K
EPAD = NW * EPT
NPAD = 10112
RPT = NPAD // NS

BLK = 1000
GRID = N // BLK


def _agg_body(h_hbm, srcs_hbm, dsts_hbm, z128_hbm, agg_hbm,
              sidx_v, didx_v, rows_v, sem, agg_s):
  c = lax.axis_index("c")
  s = lax.axis_index("s")
  wid = s * NC + c
  r0 = pl.multiple_of(s * RPT, 8)

  pltpu.sync_copy(srcs_hbm.at[wid], sidx_v)
  pltpu.sync_copy(dsts_hbm.at[wid], didx_v)
  pltpu.sync_copy(z128_hbm.at[pl.ds(r0, RPT)], agg_s.at[pl.ds(r0, RPT)])
  plsc.subcore_barrier()

  def chunk(j, carry):
    pltpu.async_copy(h_hbm.at[sidx_v.at[j]], rows_v, sem).wait()
    pltpu.sync_copy(rows_v, agg_s.at[didx_v.at[j]], add=True)
    return carry

  lax.fori_loop(0, CH, chunk, 0)
  plsc.subcore_barrier()
  pltpu.sync_copy(agg_s.at[pl.ds(r0, RPT)], agg_hbm.at[c, pl.ds(r0, RPT)])


def _cnt_body(dsts_hbm, z128_hbm, o128_hbm, cnt_hbm,
              didx_v, ones_v, cnt_s):
  c = lax.axis_index("c")
  s = lax.axis_index("s")
  wid = s * NC + c
  r0 = pl.multiple_of(s * RPT, 8)

  pltpu.sync_copy(dsts_hbm.at[wid], didx_v)
  pltpu.sync_copy(z128_hbm.at[pl.ds(r0, RPT)], cnt_s.at[pl.ds(r0, RPT)])
  pltpu.sync_copy(o128_hbm, ones_v)
  plsc.subcore_barrier()

  def chunk(j, carry):
    pltpu.sync_copy(ones_v, cnt_s.at[didx_v.at[j]], add=True)
    return carry

  lax.fori_loop(0, CH, chunk, 0)
  plsc.subcore_barrier()
  pltpu.sync_copy(cnt_s.at[pl.ds(r0, RPT)], cnt_hbm.at[c, pl.ds(r0, RPT)])


@functools.lru_cache(maxsize=None)
def _sc_kernels():
  mesh = plsc.VectorSubcoreMesh(
      core_axis_name="c", subcore_axis_name="s",
      num_cores=NC, num_subcores=NS)
  sc_agg = pl.kernel(
      _agg_body,
      out_type=jax.ShapeDtypeStruct((NC, NPAD, D), jnp.float32),
      mesh=mesh,
      scratch_types=[
          pltpu.VMEM((CH, K), jnp.int32),
          pltpu.VMEM((CH, K), jnp.int32),
          pltpu.VMEM((K, D), jnp.float32),
          pltpu.SemaphoreType.DMA,
          pltpu.VMEM_SHARED((NPAD, D), jnp.float32),
      ])
  sc_cnt = pl.kernel(
      _cnt_body,
      out_type=jax.ShapeDtypeStruct((NC, NPAD, D), jnp.float32),
      mesh=mesh,
      scratch_types=[
          pltpu.VMEM((CH, K), jnp.int32),
          pltpu.VMEM((K, D), jnp.float32),
          pltpu.VMEM_SHARED((NPAD, D), jnp.float32),
      ])
  return sc_agg, sc_cnt


def _mean(agg_ref, cnt_ref):
  inv = 1.0 / jnp.maximum(cnt_ref[0, :, 0:1] + cnt_ref[1, :, 0:1], 1.0)
  return (agg_ref[0] + agg_ref[1]) * inv


def _layer_body(agg_ref, cnt_ref, h_ref, wl_ref, bl_ref, wr_ref, out_ref):
  mean = _mean(agg_ref, cnt_ref)
  out_ref[...] = jnp.maximum(
      jnp.dot(mean, wl_ref[...], preferred_element_type=jnp.float32)
      + bl_ref[...]
      + jnp.dot(h_ref[...], wr_ref[...], preferred_element_type=jnp.float32),
      0.0)


def _final_body(agg_ref, cnt_ref, h2_ref, h1_ref, rr_ref,
                wl_ref, bl_ref, wr_ref, ws1_ref, ws2_ref, ws3_ref,
                bs_ref, alpha_ref, out_ref):
  mean = _mean(agg_ref, cnt_ref)
  h3 = jnp.maximum(
      jnp.dot(mean, wl_ref[...], preferred_element_type=jnp.float32)
      + bl_ref[...]
      + jnp.dot(h2_ref[...], wr_ref[...], preferred_element_type=jnp.float32),
      0.0)
  gnn = (jnp.dot(h1_ref[...], ws1_ref[...], preferred_element_type=jnp.float32)
         + jnp.dot(h2_ref[...], ws2_ref[...], preferred_element_type=jnp.float32)
         + jnp.dot(h3, ws3_ref[...], preferred_element_type=jnp.float32)
         + bs_ref[0, 0])
  a = 1.0 / (1.0 + jnp.exp(-alpha_ref[0, 0]))
  out_ref[...] = a * rr_ref[...] + (1.0 - a) * gnn


_agg_spec = pl.BlockSpec((NC, BLK, D), lambda i: (0, i, 0))
_cnt_spec = pl.BlockSpec((NC, BLK, D), lambda i: (0, i, 0))
_row_spec = pl.BlockSpec((BLK, D), lambda i: (i, 0))
_w_spec = pl.BlockSpec((D, D), lambda i: (0, 0))
_b_spec = pl.BlockSpec((1, D), lambda i: (0, 0))
_ws_spec = pl.BlockSpec((D, 1), lambda i: (0, 0))
_s1_spec = pl.BlockSpec((BLK, 1), lambda i: (i, 0))
_sc_spec = pl.BlockSpec((1, 1), lambda i: (0, 0))

_layer_call = pl.pallas_call(
    _layer_body,
    grid=(GRID,),
    in_specs=[_agg_spec, _cnt_spec, _row_spec, _w_spec, _b_spec, _w_spec],
    out_specs=_row_spec,
    out_shape=jax.ShapeDtypeStruct((N, D), jnp.float32),
)

_final_call = pl.pallas_call(
    _final_body,
    grid=(GRID,),
    in_specs=[_agg_spec, _cnt_spec, _row_spec, _row_spec, _s1_spec,
              _w_spec, _b_spec, _w_spec, _ws_spec, _ws_spec, _ws_spec,
              _sc_spec, _sc_spec],
    out_specs=_s1_spec,
    out_shape=jax.ShapeDtypeStruct((N, 1), jnp.float32),
)


def kernel(x, edge_index, reranker_scores, Wl0, bl0, Wr0, Wl1, bl1, Wr1,
           Wl2, bl2, Wr2, Ws, bs, alpha):
  src = edge_index[0]
  dst = edge_index[1]
  pad = EPAD - E
  srcs = jnp.concatenate([src, jnp.zeros((pad,), jnp.int32)]).reshape(NW, CH, K)
  dsts = jnp.concatenate([dst, jnp.full((pad,), N, jnp.int32)]).reshape(NW, CH, K)
  z128 = jnp.zeros((NPAD, D), jnp.float32)
  o128 = jnp.ones((K, D), jnp.float32)

  _sc_agg, _sc_cnt = _sc_kernels()
  cnt = _sc_cnt(dsts, z128, o128)
  agg0 = _sc_agg(x, srcs, dsts, z128)
  h1 = _layer_call(agg0, cnt, x, Wl0, bl0.reshape(1, D), Wr0)
  agg1 = _sc_agg(h1, srcs, dsts, z128)
  h2 = _layer_call(agg1, cnt, h1, Wl1, bl1.reshape(1, D), Wr1)
  agg2 = _sc_agg(h2, srcs, dsts, z128)
  out = _final_call(
      agg2, cnt, h2, h1, reranker_scores.reshape(N, 1),
      Wl2, bl2.reshape(1, D), Wr2,
      Ws[0:D], Ws[D:2 * D], Ws[2 * D:3 * D],
      bs.reshape(1, 1), alpha.reshape(1, 1))
  return out.reshape(N)


# speedup vs baseline: 1.5252x; 1.5194x over previous
import functools

import jax
import jax.numpy as jnp
from jax import lax
from jax.experimental import pallas as pl
from jax.experimental.pallas import tpu as pltpu
from jax.experimental.pallas import tpu_sc as plsc

N = 10000
D = 128
E = 320000

NC = 2
NS = 16
NW = NC * NS

K = 128
CH = 79
EPT = CH * K
EPAD = NW * EPT
NPAD = 10112
RPT = NPAD // NS

BLK = 1000
GRID = N // BLK


def _agg_body(h_hbm, srcs_hbm, dsts_hbm, z128_hbm, agg_hbm,
              sidx_v, didx_v, rows_v, sem, agg_s):
  c = lax.axis_index("c")
  s = lax.axis_index("s")
  wid = s * NC + c
  r0 = pl.multiple_of(s * RPT, 8)

  pltpu.sync_copy(srcs_hbm.at[wid], sidx_v)
  pltpu.sync_copy(dsts_hbm.at[wid], didx_v)
  pltpu.sync_copy(z128_hbm.at[pl.ds(r0, RPT)], agg_s.at[pl.ds(r0, RPT)])
  plsc.subcore_barrier()

  def chunk(j, carry):
    pltpu.async_copy(h_hbm.at[sidx_v.at[j]], rows_v, sem).wait()
    pltpu.sync_copy(rows_v, agg_s.at[didx_v.at[j]], add=True)
    return carry

  lax.fori_loop(0, CH, chunk, 0)
  plsc.subcore_barrier()
  pltpu.sync_copy(agg_s.at[pl.ds(r0, RPT)], agg_hbm.at[c, pl.ds(r0, RPT)])


def _cnt_body(dsts_hbm, z128_hbm, o128_hbm, cnt_hbm,
              didx_v, ones_v, cnt_s):
  c = lax.axis_index("c")
  s = lax.axis_index("s")
  wid = s * NC + c
  r0 = pl.multiple_of(s * RPT, 8)

  pltpu.sync_copy(dsts_hbm.at[wid], didx_v)
  pltpu.sync_copy(z128_hbm.at[pl.ds(r0, RPT)], cnt_s.at[pl.ds(r0, RPT)])
  pltpu.sync_copy(o128_hbm, ones_v)
  plsc.subcore_barrier()

  def chunk(j, carry):
    pltpu.sync_copy(ones_v, cnt_s.at[didx_v.at[j]], add=True)
    return carry

  lax.fori_loop(0, CH, chunk, 0)
  plsc.subcore_barrier()
  pltpu.sync_copy(cnt_s.at[pl.ds(r0, RPT)], cnt_hbm.at[c, pl.ds(r0, RPT)])


@functools.lru_cache(maxsize=None)
def _sc_kernels():
  mesh = plsc.VectorSubcoreMesh(
      core_axis_name="c", subcore_axis_name="s",
      num_cores=NC, num_subcores=NS)
  sc_agg = pl.kernel(
      _agg_body,
      out_type=jax.ShapeDtypeStruct((NC, NPAD, D), jnp.float32),
      mesh=mesh,
      scratch_types=[
          pltpu.VMEM((CH, K), jnp.int32),
          pltpu.VMEM((CH, K), jnp.int32),
          pltpu.VMEM((K, D), jnp.float32),
          pltpu.SemaphoreType.DMA,
          pltpu.VMEM_SHARED((NPAD, D), jnp.float32),
      ])
  sc_cnt = pl.kernel(
      _cnt_body,
      out_type=jax.ShapeDtypeStruct((NC, NPAD, D), jnp.float32),
      mesh=mesh,
      scratch_types=[
          pltpu.VMEM((CH, K), jnp.int32),
          pltpu.VMEM((K, D), jnp.float32),
          pltpu.VMEM_SHARED((NPAD, D), jnp.float32),
      ])
  return sc_agg, sc_cnt


def _mean(agg_ref, cnt_ref):
  inv = 1.0 / jnp.maximum(cnt_ref[0, :, 0:1] + cnt_ref[1, :, 0:1], 1.0)
  return (agg_ref[0] + agg_ref[1]) * inv


def _layer_body(agg_ref, cnt_ref, h_ref, wl_ref, bl_ref, wr_ref, out_ref):
  mean = _mean(agg_ref, cnt_ref)
  out_ref[...] = jnp.maximum(
      jnp.dot(mean, wl_ref[...], preferred_element_type=jnp.float32)
      + bl_ref[...]
      + jnp.dot(h_ref[...], wr_ref[...], preferred_element_type=jnp.float32),
      0.0)


def _final_body(agg_ref, cnt_ref, h2_ref, h1_ref, rr_ref,
                wl_ref, bl_ref, wr_ref, ws1_ref, ws2_ref, ws3_ref,
                bs_ref, alpha_ref, out_ref):
  mean = _mean(agg_ref, cnt_ref)
  h3 = jnp.maximum(
      jnp.dot(mean, wl_ref[...], preferred_element_type=jnp.float32)
      + bl_ref[...]
      + jnp.dot(h2_ref[...], wr_ref[...], preferred_element_type=jnp.float32),
      0.0)
  gnn = (jnp.dot(h1_ref[...], ws1_ref[...], preferred_element_type=jnp.float32)
         + jnp.dot(h2_ref[...], ws2_ref[...], preferred_element_type=jnp.float32)
         + jnp.dot(h3, ws3_ref[...], preferred_element_type=jnp.float32)
         + bs_ref[0, 0])
  a = 1.0 / (1.0 + jnp.exp(-alpha_ref[0, 0]))
  out_ref[...] = a * rr_ref[...] + (1.0 - a) * gnn


_agg_spec = pl.BlockSpec((NC, BLK, D), lambda i: (0, i, 0))
_cnt_spec = pl.BlockSpec((NC, BLK, D), lambda i: (0, i, 0))
_row_spec = pl.BlockSpec((BLK, D), lambda i: (i, 0))
_w_spec = pl.BlockSpec((D, D), lambda i: (0, 0))
_b_spec = pl.BlockSpec((1, D), lambda i: (0, 0))
_ws_spec = pl.BlockSpec((D, 1), lambda i: (0, 0))
_s1_spec = pl.BlockSpec((BLK, 1), lambda i: (i, 0))
_sc_spec = pl.BlockSpec((1, 1), lambda i: (0, 0))

_layer_call = pl.pallas_call(
    _layer_body,
    grid=(GRID,),
    in_specs=[_agg_spec, _cnt_spec, _row_spec, _w_spec, _b_spec, _w_spec],
    out_specs=_row_spec,
    out_shape=jax.ShapeDtypeStruct((N, D), jnp.float32),
)

_final_call = pl.pallas_call(
    _final_body,
    grid=(GRID,),
    in_specs=[_agg_spec, _cnt_spec, _row_spec, _row_spec, _s1_spec,
              _w_spec, _b_spec, _w_spec, _ws_spec, _ws_spec, _ws_spec,
              _sc_spec, _sc_spec],
    out_specs=_s1_spec,
    out_shape=jax.ShapeDtypeStruct((N, 1), jnp.float32),
)


def kernel(x, edge_index, reranker_scores, Wl0, bl0, Wr0, Wl1, bl1, Wr1,
           Wl2, bl2, Wr2, Ws, bs, alpha):
  src = edge_index[0]
  dst = edge_index[1]
  pad = EPAD - E
  srcs = jnp.concatenate([src, jnp.zeros((pad,), jnp.int32)]).reshape(NW, CH, K)
  dsts = jnp.concatenate([dst, jnp.full((pad,), N, jnp.int32)]).reshape(NW, CH, K)
  z128 = jnp.zeros((NPAD, D), jnp.float32)
  o128 = jnp.ones((K, D), jnp.float32)

  _sc_agg, _sc_cnt = _sc_kernels()
  cnt = _sc_cnt(dsts, z128, o128)
  agg0 = _sc_agg(x, srcs, dsts, z128)
  h1 = _layer_call(agg0, cnt, x, Wl0, bl0.reshape(1, D), Wr0)
  agg1 = _sc_agg(h1, srcs, dsts, z128)
  h2 = _layer_call(agg1, cnt, h1, Wl1, bl1.reshape(1, D), Wr1)
  agg2 = _sc_agg(h2, srcs, dsts, z128)
  out = _final_call(
      agg2, cnt, h2, h1, reranker_scores.reshape(N, 1),
      Wl2, bl2.reshape(1, D), Wr2,
      Ws[0:D], Ws[D:2 * D], Ws[2 * D:3 * D],
      bs.reshape(1, 1), alpha.reshape(1, 1))
  return out.reshape(N)


# speedup vs baseline: 1.6872x; 1.1062x over previous
import functools

import jax
import jax.numpy as jnp
from jax import lax
from jax.experimental import pallas as pl
from jax.experimental.pallas import tpu as pltpu
from jax.experimental.pallas import tpu_sc as plsc

N = 10000
D = 128
E = 320000

NC = 2
NS = 16
NW = NC * NS

K = 128
CH = 79
EPT = CH * K
EPAD = NW * EPT
NPAD = 10112
RPT = NPAD // NS

BLK = 1000
GRID = N // BLK


def _agg_body(h_hbm, srcs_hbm, dsts_hbm, z128_hbm, agg_hbm,
              sidx_v, didx_v, rows_v, sem, agg_s):
  c = lax.axis_index("c")
  s = lax.axis_index("s")
  wid = s * NC + c
  r0 = pl.multiple_of(s * RPT, 8)

  pltpu.sync_copy(srcs_hbm.at[wid], sidx_v)
  pltpu.sync_copy(dsts_hbm.at[wid], didx_v)
  pltpu.sync_copy(z128_hbm.at[pl.ds(r0, RPT)], agg_s.at[pl.ds(r0, RPT)])
  plsc.subcore_barrier()

  def chunk(j, carry):
    pltpu.async_copy(h_hbm.at[sidx_v.at[j]], rows_v, sem).wait()
    pltpu.sync_copy(rows_v, agg_s.at[didx_v.at[j]], add=True)
    return carry

  lax.fori_loop(0, CH, chunk, 0)
  plsc.subcore_barrier()
  pltpu.sync_copy(agg_s.at[pl.ds(r0, RPT)], agg_hbm.at[c, pl.ds(r0, RPT)])


def _cnt_body(dsts_hbm, z128_hbm, o128_hbm, cnt_hbm,
              didx_v, ones_v, cnt_s):
  c = lax.axis_index("c")
  s = lax.axis_index("s")
  wid = s * NC + c
  r0 = pl.multiple_of(s * RPT, 8)

  pltpu.sync_copy(dsts_hbm.at[wid], didx_v)
  pltpu.sync_copy(z128_hbm.at[pl.ds(r0, RPT)], cnt_s.at[pl.ds(r0, RPT)])
  pltpu.sync_copy(o128_hbm, ones_v)
  plsc.subcore_barrier()

  def chunk(j, carry):
    pltpu.sync_copy(ones_v, cnt_s.at[didx_v.at[j]], add=True)
    return carry

  lax.fori_loop(0, CH, chunk, 0)
  plsc.subcore_barrier()
  pltpu.sync_copy(cnt_s.at[pl.ds(r0, RPT)], cnt_hbm.at[c, pl.ds(r0, RPT)])


@functools.lru_cache(maxsize=None)
def _sc_kernels():
  mesh = plsc.VectorSubcoreMesh(
      core_axis_name="c", subcore_axis_name="s",
      num_cores=NC, num_subcores=NS)
  sc_agg = pl.kernel(
      _agg_body,
      out_type=jax.ShapeDtypeStruct((NC, NPAD, D), jnp.float32),
      mesh=mesh,
      scratch_types=[
          pltpu.VMEM((CH, K), jnp.int32),
          pltpu.VMEM((CH, K), jnp.int32),
          pltpu.VMEM((K, D), jnp.float32),
          pltpu.SemaphoreType.DMA,
          pltpu.VMEM_SHARED((NPAD, D), jnp.float32),
      ])
  sc_cnt = pl.kernel(
      _cnt_body,
      out_type=jax.ShapeDtypeStruct((NC, NPAD, D), jnp.float32),
      mesh=mesh,
      scratch_types=[
          pltpu.VMEM((CH, K), jnp.int32),
          pltpu.VMEM((K, D), jnp.float32),
          pltpu.VMEM_SHARED((NPAD, D), jnp.float32),
      ])
  return sc_agg, sc_cnt


def _mean(agg_ref, cnt_ref):
  inv = 1.0 / jnp.maximum(cnt_ref[0, :, 0:1] + cnt_ref[1, :, 0:1], 1.0)
  return (agg_ref[0] + agg_ref[1]) * inv


def _layer_body(agg_ref, cnt_ref, h_ref, wl_ref, bl_ref, wr_ref, out_ref):
  mean = _mean(agg_ref, cnt_ref)
  out_ref[...] = jnp.maximum(
      jnp.dot(mean, wl_ref[...], preferred_element_type=jnp.float32)
      + bl_ref[...]
      + jnp.dot(h_ref[...], wr_ref[...], preferred_element_type=jnp.float32),
      0.0)


def _final_body(agg_ref, cnt_ref, h2_ref, h1_ref, rr_ref,
                wl_ref, bl_ref, wr_ref, ws1_ref, ws2_ref, ws3_ref,
                bs_ref, alpha_ref, out_ref):
  mean = _mean(agg_ref, cnt_ref)
  h3 = jnp.maximum(
      jnp.dot(mean, wl_ref[...], preferred_element_type=jnp.float32)
      + bl_ref[...]
      + jnp.dot(h2_ref[...], wr_ref[...], preferred_element_type=jnp.float32),
      0.0)
  gnn = (jnp.dot(h1_ref[...], ws1_ref[...], preferred_element_type=jnp.float32)
         + jnp.dot(h2_ref[...], ws2_ref[...], preferred_element_type=jnp.float32)
         + jnp.dot(h3, ws3_ref[...], preferred_element_type=jnp.float32)
         + bs_ref[0, 0])
  a = 1.0 / (1.0 + jnp.exp(-alpha_ref[0, 0]))
  out_ref[...] = a * rr_ref[...] + (1.0 - a) * gnn


_agg_spec = pl.BlockSpec((NC, BLK, D), lambda i: (0, i, 0))
_cnt_spec = pl.BlockSpec((NC, BLK, D), lambda i: (0, i, 0))
_row_spec = pl.BlockSpec((BLK, D), lambda i: (i, 0))
_w_spec = pl.BlockSpec((D, D), lambda i: (0, 0))
_b_spec = pl.BlockSpec((1, D), lambda i: (0, 0))
_ws_spec = pl.BlockSpec((D, 1), lambda i: (0, 0))
_s1_spec = pl.BlockSpec((BLK, 1), lambda i: (i, 0))
_sc_spec = pl.BlockSpec((1, 1), lambda i: (0, 0))

_layer_call = pl.pallas_call(
    _layer_body,
    grid=(GRID,),
    in_specs=[_agg_spec, _cnt_spec, _row_spec, _w_spec, _b_spec, _w_spec],
    out_specs=_row_spec,
    out_shape=jax.ShapeDtypeStruct((N, D), jnp.float32),
)

_final_call = pl.pallas_call(
    _final_body,
    grid=(GRID,),
    in_specs=[_agg_spec, _cnt_spec, _row_spec, _row_spec, _s1_spec,
              _w_spec, _b_spec, _w_spec, _ws_spec, _ws_spec, _ws_spec,
              _sc_spec, _sc_spec],
    out_specs=_s1_spec,
    out_shape=jax.ShapeDtypeStruct((N, 1), jnp.float32),
)


def kernel(x, edge_index, reranker_scores, Wl0, bl0, Wr0, Wl1, bl1, Wr1,
           Wl2, bl2, Wr2, Ws, bs, alpha):
  src = edge_index[0]
  dst = edge_index[1]
  pad = EPAD - E
  pad_dst = N + (jnp.arange(pad, dtype=jnp.int32) % (NPAD - N))
  srcs = (jnp.concatenate([src, jnp.zeros((pad,), jnp.int32)])
          .reshape(EPT, NW).T.reshape(NW, CH, K))
  dsts = (jnp.concatenate([dst, pad_dst])
          .reshape(EPT, NW).T.reshape(NW, CH, K))
  z128 = jnp.zeros((NPAD, D), jnp.float32)
  o128 = jnp.ones((K, D), jnp.float32)

  _sc_agg, _sc_cnt = _sc_kernels()
  cnt = _sc_cnt(dsts, z128, o128)
  agg0 = _sc_agg(x, srcs, dsts, z128)
  h1 = _layer_call(agg0, cnt, x, Wl0, bl0.reshape(1, D), Wr0)
  agg1 = _sc_agg(h1, srcs, dsts, z128)
  h2 = _layer_call(agg1, cnt, h1, Wl1, bl1.reshape(1, D), Wr1)
  agg2 = _sc_agg(h2, srcs, dsts, z128)
  out = _final_call(
      agg2, cnt, h2, h1, reranker_scores.reshape(N, 1),
      Wl2, bl2.reshape(1, D), Wr2,
      Ws[0:D], Ws[D:2 * D], Ws[2 * D:3 * D],
      bs.reshape(1, 1), alpha.reshape(1, 1))
  return out.reshape(N)
